# baseline (speedup 1.0000x reference)
import functools

import jax
import jax.numpy as jnp
from jax import lax
from jax.experimental import pallas as pl
from jax.experimental.pallas import tpu as pltpu
from jax.experimental.pallas import tpu_sc as plsc

D_MODEL = 768
B_TOTAL = 1024 * 200

_info = plsc.get_sparse_core_info()
_NC, _NS = _info.num_cores, _info.num_subcores
_NW = _NC * _NS
_B_PER_W = B_TOTAL // _NW
_NB = 4
_CHUNK = 40
_N_CHUNKS = _B_PER_W // _CHUNK


@functools.partial(
    pl.kernel,
    mesh=plsc.VectorSubcoreMesh(core_axis_name="c", subcore_axis_name="s"),
    out_type=jax.ShapeDtypeStruct((B_TOTAL, D_MODEL), jnp.float32),
    scratch_types=(
        [pltpu.VMEM((_B_PER_W,), jnp.int32)]
        + [pltpu.VMEM((_CHUNK, D_MODEL), jnp.float32) for _ in range(_NB)]
        + [pltpu.SemaphoreType.DMA for _ in range(2 * _NB)]
    ),
)
def _gather_kernel(idx_hbm, table_hbm, out_hbm, idx_v, *scratch):
    bufs = scratch[:_NB]
    gsems = scratch[_NB : 2 * _NB]
    osems = scratch[2 * _NB : 3 * _NB]

    wid = lax.axis_index("s") * _NC + lax.axis_index("c")
    base = wid * _B_PER_W
    pltpu.sync_copy(idx_hbm.at[pl.ds(base, _B_PER_W)], idx_v)

    def off_of(c):
        return pl.multiple_of(c * _CHUNK, 8)

    def gather(c, b):
        return pltpu.make_async_copy(
            table_hbm.at[idx_v.at[pl.ds(off_of(c), _CHUNK)]], bufs[b], gsems[b]
        )

    def out(c, b):
        return pltpu.make_async_copy(
            bufs[b], out_hbm.at[pl.ds(base + off_of(c), _CHUNK)], osems[b]
        )

    gather(0, 0).start()
    for c in range(_NB - 1):
        gather(c, c).wait()
        out(c, c).start()
        gather(c + 1, c + 1).start()

    def group(g, carry):
        for j in range(_NB):
            c = (_NB - 1) + g * _NB + j
            b = (_NB - 1 + j) % _NB
            bn = (b + 1) % _NB
            gather(c, b).wait()
            out(c, b).start()
            out(c + 1 - _NB, bn).wait()
            gather(c + 1, bn).start()
        return carry

    lax.fori_loop(0, (_N_CHUNKS - _NB) // _NB, group, 0)

    c_last = _N_CHUNKS - 1
    b_last = c_last % _NB
    gather(c_last, b_last).wait()
    out(c_last, b_last).start()
    for k in range(_NB):
        c = _N_CHUNKS - _NB + k
        out(c, c % _NB).wait()


def kernel(token_ids, embeddings):
    idx = token_ids.reshape(-1).astype(jnp.int32)
    out = _gather_kernel(idx, embeddings)
    return out.reshape(token_ids.shape + (D_MODEL,))


# speedup vs baseline: 1.0108x; 1.0108x over previous
import functools

import jax
import jax.numpy as jnp
from jax import lax
from jax.experimental import pallas as pl
from jax.experimental.pallas import tpu as pltpu
from jax.experimental.pallas import tpu_sc as plsc

D_MODEL = 768
B_TOTAL = 1024 * 200

_info = plsc.get_sparse_core_info()
_NC, _NS = _info.num_cores, _info.num_subcores
_NW = _NC * _NS
_B_PER_W = B_TOTAL // _NW
_NB = 2
_CHUNK = 80
_N_CHUNKS = _B_PER_W // _CHUNK


@functools.partial(
    pl.kernel,
    mesh=plsc.VectorSubcoreMesh(core_axis_name="c", subcore_axis_name="s"),
    out_type=jax.ShapeDtypeStruct((B_TOTAL, D_MODEL), jnp.float32),
    scratch_types=(
        [pltpu.VMEM((_B_PER_W,), jnp.int32)]
        + [pltpu.VMEM((_CHUNK, D_MODEL), jnp.float32) for _ in range(_NB)]
        + [pltpu.SemaphoreType.DMA for _ in range(2 * _NB)]
    ),
)
def _gather_kernel(idx_hbm, table_hbm, out_hbm, idx_v, *scratch):
    bufs = scratch[:_NB]
    gsems = scratch[_NB : 2 * _NB]
    osems = scratch[2 * _NB : 3 * _NB]

    wid = lax.axis_index("s") * _NC + lax.axis_index("c")
    base = wid * _B_PER_W
    pltpu.sync_copy(idx_hbm.at[pl.ds(base, _B_PER_W)], idx_v)

    def off_of(c):
        return pl.multiple_of(c * _CHUNK, 8)

    def gather(c, b):
        return pltpu.make_async_copy(
            table_hbm.at[idx_v.at[pl.ds(off_of(c), _CHUNK)]], bufs[b], gsems[b]
        )

    def out(c, b):
        return pltpu.make_async_copy(
            bufs[b], out_hbm.at[pl.ds(base + off_of(c), _CHUNK)], osems[b]
        )

    gather(0, 0).start()
    for c in range(_NB - 1):
        gather(c, c).wait()
        out(c, c).start()
        gather(c + 1, c + 1).start()

    def group(g, carry):
        for j in range(_NB):
            c = (_NB - 1) + g * _NB + j
            b = (_NB - 1 + j) % _NB
            bn = (b + 1) % _NB
            gather(c, b).wait()
            out(c, b).start()
            out(c + 1 - _NB, bn).wait()
            gather(c + 1, bn).start()
        return carry

    lax.fori_loop(0, (_N_CHUNKS - _NB) // _NB, group, 0)

    c_last = _N_CHUNKS - 1
    b_last = c_last % _NB
    gather(c_last, b_last).wait()
    out(c_last, b_last).start()
    for k in range(_NB):
        c = _N_CHUNKS - _NB + k
        out(c, c % _NB).wait()


def kernel(token_ids, embeddings):
    idx = token_ids.reshape(-1).astype(jnp.int32)
    out = _gather_kernel(idx, embeddings)
    return out.reshape(token_ids.shape + (D_MODEL,))
